# baseline (speedup 1.0000x reference)
import jax
import jax.numpy as jnp
from jax.experimental import pallas as pl
from jax.experimental.pallas import tpu as pltpu

_NCHUNK = 32


def _copy_body(src_ref, out_ref, sems):
    rows = src_ref.shape[0]
    chunk = rows // _NCHUNK
    for i in range(_NCHUNK):
        pltpu.make_async_copy(
            src_ref.at[pl.ds(i * chunk, chunk)],
            out_ref.at[pl.ds(i * chunk, chunk)],
            sems.at[i],
        ).start()
    for i in range(_NCHUNK):
        pltpu.make_async_copy(
            src_ref.at[pl.ds(i * chunk, chunk)],
            out_ref.at[pl.ds(i * chunk, chunk)],
            sems.at[i],
        ).wait()


def kernel(x, pos_emb):
    S = x.shape[0]
    D = pos_emb.shape[1]
    src = pos_emb[:S]
    out = pl.pallas_call(
        _copy_body,
        in_specs=[pl.BlockSpec(memory_space=pltpu.MemorySpace.HBM)],
        out_specs=pl.BlockSpec(memory_space=pltpu.MemorySpace.HBM),
        out_shape=jax.ShapeDtypeStruct((S, D), jnp.float32),
        scratch_shapes=[pltpu.SemaphoreType.DMA((_NCHUNK,))],
    )(src)
    return out.reshape(S, 1, D)


# speedup vs baseline: 15.9939x; 15.9939x over previous
import jax
import jax.numpy as jnp
from jax.experimental import pallas as pl
from jax.experimental.pallas import tpu as pltpu

_BLOCK_ROWS = 512


def _copy_body(src_ref, out_ref):
    out_ref[...] = src_ref[...]


def kernel(x, pos_emb):
    S = x.shape[0]
    D = pos_emb.shape[1]
    src = pos_emb[:S]
    out = pl.pallas_call(
        _copy_body,
        grid=(S // _BLOCK_ROWS,),
        in_specs=[pl.BlockSpec((_BLOCK_ROWS, D), lambda i: (i, 0))],
        out_specs=pl.BlockSpec((_BLOCK_ROWS, D), lambda i: (i, 0)),
        out_shape=jax.ShapeDtypeStruct((S, D), jnp.float32),
    )(src)
    return out.reshape(S, 1, D)


# speedup vs baseline: 16.2808x; 1.0179x over previous
import jax
import jax.numpy as jnp
from jax.experimental import pallas as pl
from jax.experimental.pallas import tpu as pltpu

_BLOCK_ROWS = 1024


def _copy_body(src_ref, out_ref):
    out_ref[...] = src_ref[...]


def kernel(x, pos_emb):
    S = x.shape[0]
    D = pos_emb.shape[1]
    src = pos_emb[:S]
    out = pl.pallas_call(
        _copy_body,
        grid=(S // _BLOCK_ROWS,),
        in_specs=[pl.BlockSpec((_BLOCK_ROWS, D), lambda i: (i, 0))],
        out_specs=pl.BlockSpec((_BLOCK_ROWS, D), lambda i: (i, 0)),
        out_shape=jax.ShapeDtypeStruct((S, D), jnp.float32),
    )(src)
    return out.reshape(S, 1, D)
